# baseline (speedup 1.0000x reference)
import jax
import jax.numpy as jnp
from jax.experimental import pallas as pl
from jax.experimental.pallas import tpu as pltpu

_EPS = 1e-09
_W = 5000
_BLOCK_B = 512
_THIRD = 1.0 / 3.0


def _body(x_ref, o_ref, m_ref):
    for it in range(4):
        src = x_ref if it == 0 else o_ref
        dst = o_ref
        m_ref[1:4999] = jnp.minimum(
            jnp.minimum(src[0:4998], src[1:4999]), src[2:5000]
        )
        m = m_ref[1:4999]
        pmin = jnp.min(m, axis=0, keepdims=True)
        pmax = jnp.max(m, axis=0, keepdims=True)
        rng = pmax - pmin
        inv = 1.0 / rng
        q = pmin - _EPS * rng
        m_ref[0:1] = q
        m_ref[4999:5002] = jnp.broadcast_to(q, (3, q.shape[1]))
        s3 = (m_ref[0:5000] + m_ref[1:5001] + m_ref[2:5002]) * (inv * _THIRD)
        if it == 3:
            dst[...] = s3 + (_EPS - pmin * inv)
        else:
            dst[...] = s3


def _retile_body(x_ref, o_ref):
    o_ref[...] = x_ref[...].reshape(x_ref.shape[0], 8, 128)


def kernel(x):
    x = x.reshape(-1, _W).astype(jnp.float32)
    n = x.shape[0]
    xt = x.T
    out = pl.pallas_call(
        _body,
        grid=(n // _BLOCK_B,),
        in_specs=[pl.BlockSpec((_W, _BLOCK_B), lambda i: (0, i))],
        out_specs=pl.BlockSpec((_W, _BLOCK_B), lambda i: (0, i)),
        out_shape=jax.ShapeDtypeStruct((_W, n), jnp.float32),
        scratch_shapes=[
            pltpu.VMEM((_W + 2, _BLOCK_B), jnp.float32),
        ],
        compiler_params=pltpu.CompilerParams(
            dimension_semantics=("parallel",)
        ),
    )(xt)
    nc = n // 128
    sr = 1000
    re = pl.pallas_call(
        _retile_body,
        grid=(nc // 8, _W // sr),
        in_specs=[pl.BlockSpec((sr, 1024), lambda i, j: (j, i))],
        out_specs=pl.BlockSpec((sr, 8, 128), lambda i, j: (j, i, 0)),
        out_shape=jax.ShapeDtypeStruct((_W, nc, 128), jnp.float32),
        compiler_params=pltpu.CompilerParams(
            dimension_semantics=("parallel", "arbitrary")
        ),
    )(out)
    return re.transpose(1, 2, 0).reshape(n, 1, 50, 100)


# speedup vs baseline: 1.0054x; 1.0054x over previous
import jax
import jax.numpy as jnp
from jax.experimental import pallas as pl
from jax.experimental.pallas import tpu as pltpu

_EPS = 1e-09
_W = 5000
_BLOCK_B = 256
_THIRD = 1.0 / 3.0


def _body(x_ref, o_ref, m_ref, xb_ref):
    for it in range(4):
        src = x_ref if it == 0 else xb_ref
        dst = o_ref if it == 3 else xb_ref
        m_ref[1:4999] = jnp.minimum(
            jnp.minimum(src[0:4998], src[1:4999]), src[2:5000]
        )
        m = m_ref[1:4999]
        pmin = jnp.min(m, axis=0, keepdims=True)
        pmax = jnp.max(m, axis=0, keepdims=True)
        rng = pmax - pmin
        inv = 1.0 / rng
        q = pmin - _EPS * rng
        m_ref[0:1] = q
        m_ref[4999:5002] = jnp.broadcast_to(q, (3, q.shape[1]))
        s3 = (m_ref[0:5000] + m_ref[1:5001] + m_ref[2:5002]) * (inv * _THIRD)
        if it == 3:
            dst[...] = s3 + (_EPS - pmin * inv)
        else:
            dst[...] = s3


def _retile_body(x_ref, o_ref):
    o_ref[...] = x_ref[...].reshape(x_ref.shape[0], 8, 128)


def kernel(x):
    x = x.reshape(-1, _W).astype(jnp.float32)
    n = x.shape[0]
    xt = x.T
    out = pl.pallas_call(
        _body,
        grid=(n // _BLOCK_B,),
        in_specs=[pl.BlockSpec((_W, _BLOCK_B), lambda i: (0, i))],
        out_specs=pl.BlockSpec((_W, _BLOCK_B), lambda i: (0, i)),
        out_shape=jax.ShapeDtypeStruct((_W, n), jnp.float32),
        scratch_shapes=[
            pltpu.VMEM((_W + 2, _BLOCK_B), jnp.float32),
            pltpu.VMEM((_W, _BLOCK_B), jnp.float32),
        ],
        compiler_params=pltpu.CompilerParams(
            dimension_semantics=("parallel",)
        ),
    )(xt)
    nc = n // 128
    sr = 1000
    re = pl.pallas_call(
        _retile_body,
        grid=(nc // 8, _W // sr),
        in_specs=[pl.BlockSpec((sr, 1024), lambda i, j: (j, i))],
        out_specs=pl.BlockSpec((sr, 8, 128), lambda i, j: (j, i, 0)),
        out_shape=jax.ShapeDtypeStruct((_W, nc, 128), jnp.float32),
        compiler_params=pltpu.CompilerParams(
            dimension_semantics=("parallel", "arbitrary")
        ),
    )(out)
    return re.transpose(1, 2, 0).reshape(n, 1, 50, 100)
